# baseline (speedup 1.0000x reference)
import functools

import jax
import jax.numpy as jnp
from jax.experimental import pallas as pl
from jax.experimental.pallas import tpu as pltpu

_SIGMA = 0.4
_INV2S2 = 1.0 / (2.0 * _SIGMA * _SIGMA)


def _ga_kernel(in_len_ref, out_len_ref, a_ref, o_ref, *, blk, t_in):
    b = pl.program_id(0)
    k = pl.program_id(1)
    ti = in_len_ref[b].astype(jnp.float32)
    to = out_len_ref[b].astype(jnp.float32)
    r = to / ti
    base = (k * blk).astype(jnp.float32)

    a = a_ref[0]
    i = jax.lax.broadcasted_iota(jnp.int32, (blk, t_in), 0).astype(jnp.float32) + base
    j = jax.lax.broadcasted_iota(jnp.int32, (blk, t_in), 1).astype(jnp.float32)
    d = i - j * r
    w = 1.0 - jnp.exp(d * d * (-_INV2S2))
    valid = (i < to) & (j < ti)
    total = jnp.sum(jnp.where(valid, a * w, 0.0))

    @pl.when(k == 0)
    def _init():
        o_ref[...] = jnp.zeros_like(o_ref)

    o_ref[...] += jnp.full(o_ref.shape, total, jnp.float32)


def kernel(alignments, input_lengths, output_lengths):
    B, T_out, T_in = alignments.shape
    K = 2
    blk = T_out // K
    grid_spec = pltpu.PrefetchScalarGridSpec(
        num_scalar_prefetch=2,
        grid=(B, K),
        in_specs=[
            pl.BlockSpec((1, blk, T_in), lambda b, k, *_: (b, k, 0)),
        ],
        out_specs=pl.BlockSpec((1, 8, 128), lambda b, k, *_: (b, 0, 0)),
    )
    partials = pl.pallas_call(
        functools.partial(_ga_kernel, blk=blk, t_in=T_in),
        grid_spec=grid_spec,
        out_shape=jax.ShapeDtypeStruct((B, 8, 128), jnp.float32),
        compiler_params=pltpu.CompilerParams(
            dimension_semantics=("parallel", "arbitrary"),
        ),
    )(input_lengths, output_lengths, alignments)
    return jnp.sum(partials[:, 0, 0]) / B


# speedup vs baseline: 2.0431x; 2.0431x over previous
import functools
import math

import jax
import jax.numpy as jnp
from jax.experimental import pallas as pl
from jax.experimental.pallas import tpu as pltpu

_SIGMA = 0.4
_C = math.sqrt(math.log2(math.e) / (2.0 * _SIGMA * _SIGMA))


def _ga_kernel(in_len_ref, out_len_ref, a_ref, o_ref, is_ref, *, blk, t_in):
    b = pl.program_id(0)
    ti = in_len_ref[b].astype(jnp.float32)
    to = out_len_ref[b].astype(jnp.float32)
    rc = to / ti * _C

    @pl.when(b == 0)
    def _init_scaled_iota():
        i2 = jax.lax.broadcasted_iota(jnp.int32, (blk, t_in), 0)
        is_ref[...] = i2.astype(jnp.float32) * _C

    a = a_ref[0]
    j_row = jax.lax.broadcasted_iota(jnp.int32, (1, t_in), 1).astype(jnp.float32)
    es_row = j_row * rc
    mj = (j_row < ti).astype(jnp.float32)

    isc = is_ref[...]
    u = isc - es_row
    g = jnp.exp2(u * (es_row - isc))
    am = a * mj
    q = am - am * g
    rs = jnp.sum(q, axis=1, keepdims=True)

    i_col = jax.lax.broadcasted_iota(jnp.int32, (blk, 1), 0).astype(jnp.float32)
    mi = (i_col < to).astype(jnp.float32)
    total = jnp.sum(rs * mi)
    o_ref[0] = jnp.full((8, 128), total, jnp.float32)


def kernel(alignments, input_lengths, output_lengths):
    B, T_out, T_in = alignments.shape
    grid_spec = pltpu.PrefetchScalarGridSpec(
        num_scalar_prefetch=2,
        grid=(B,),
        in_specs=[
            pl.BlockSpec((1, T_out, T_in), lambda b, *_: (b, 0, 0)),
        ],
        out_specs=pl.BlockSpec((1, 8, 128), lambda b, *_: (b, 0, 0)),
        scratch_shapes=[pltpu.VMEM((T_out, T_in), jnp.float32)],
    )
    partials = pl.pallas_call(
        functools.partial(_ga_kernel, blk=T_out, t_in=T_in),
        grid_spec=grid_spec,
        out_shape=jax.ShapeDtypeStruct((B, 8, 128), jnp.float32),
        compiler_params=pltpu.CompilerParams(
            dimension_semantics=("arbitrary",),
        ),
    )(input_lengths, output_lengths, alignments)
    return jnp.sum(partials[:, 0, 0]) / B


# speedup vs baseline: 2.0498x; 1.0033x over previous
import functools
import math

import jax
import jax.numpy as jnp
from jax.experimental import pallas as pl
from jax.experimental.pallas import tpu as pltpu

_SIGMA = 0.4
_C = math.sqrt(math.log2(math.e) / (2.0 * _SIGMA * _SIGMA))


def _ga_kernel(in_len_ref, out_len_ref, a_ref, o_ref, is_ref, *, blk, t_in):
    b = pl.program_id(0)
    ti = in_len_ref[b].astype(jnp.float32)
    to = out_len_ref[b].astype(jnp.float32)
    rc = to / ti * _C

    @pl.when(b == 0)
    def _init_scaled_iota():
        i2 = jax.lax.broadcasted_iota(jnp.int32, (blk, 128), 0)
        is_ref[...] = i2.astype(jnp.float32) * _C

    a = a_ref[0]
    j_row = jax.lax.broadcasted_iota(jnp.int32, (1, t_in), 1).astype(jnp.float32)
    es_row = j_row * rc
    mj = (j_row < ti).astype(jnp.float32)

    isc = pltpu.repeat(is_ref[...], t_in // 128, axis=1)
    u = isc - es_row
    g = jnp.exp2(u * (es_row - isc))
    am = a * mj
    q = am - am * g
    rs = jnp.sum(q, axis=1, keepdims=True)

    i_col = jax.lax.broadcasted_iota(jnp.int32, (blk, 1), 0).astype(jnp.float32)
    mi = (i_col < to).astype(jnp.float32)
    total = jnp.sum(rs * mi)
    o_ref[0] = jnp.full((8, 128), total, jnp.float32)


def kernel(alignments, input_lengths, output_lengths):
    B, T_out, T_in = alignments.shape
    grid_spec = pltpu.PrefetchScalarGridSpec(
        num_scalar_prefetch=2,
        grid=(B,),
        in_specs=[
            pl.BlockSpec((1, T_out, T_in), lambda b, *_: (b, 0, 0)),
        ],
        out_specs=pl.BlockSpec((1, 8, 128), lambda b, *_: (b, 0, 0)),
        scratch_shapes=[pltpu.VMEM((T_out, 128), jnp.float32)],
    )
    partials = pl.pallas_call(
        functools.partial(_ga_kernel, blk=T_out, t_in=T_in),
        grid_spec=grid_spec,
        out_shape=jax.ShapeDtypeStruct((B, 8, 128), jnp.float32),
        compiler_params=pltpu.CompilerParams(
            dimension_semantics=("arbitrary",),
        ),
    )(input_lengths, output_lengths, alignments)
    return jnp.sum(partials[:, 0, 0]) / B


# speedup vs baseline: 2.3779x; 1.1601x over previous
import functools
import math

import jax
import jax.numpy as jnp
from jax.experimental import pallas as pl
from jax.experimental.pallas import tpu as pltpu

_SIGMA = 0.4
_C = math.sqrt(math.log2(math.e) / (2.0 * _SIGMA * _SIGMA))


def _ga_kernel(in_len_ref, out_len_ref, a_ref, o_ref, is_ref, *, blk, t_in):
    b = pl.program_id(0)

    @pl.when(b == 0)
    def _init_scaled_iota():
        i2 = jax.lax.broadcasted_iota(jnp.int32, (blk, 128), 0)
        is_ref[...] = i2.astype(jnp.float32) * _C

    isc = pltpu.repeat(is_ref[...], t_in // 128, axis=1)
    j_row = jax.lax.broadcasted_iota(jnp.int32, (1, t_in), 1).astype(jnp.float32)
    i_col = jax.lax.broadcasted_iota(jnp.int32, (blk, 1), 0).astype(jnp.float32)

    for s in range(a_ref.shape[0]):
        ti = in_len_ref[2 * b + s].astype(jnp.float32)
        to = out_len_ref[2 * b + s].astype(jnp.float32)
        rc = to / ti * _C

        a = a_ref[s]
        es_row = j_row * rc
        mj = (j_row < ti).astype(jnp.float32)

        u = isc - es_row
        g = jnp.exp2(u * (es_row - isc))
        am = a * mj
        q = am - am * g
        rs = jnp.sum(q, axis=1, keepdims=True)

        mi = (i_col < to).astype(jnp.float32)
        total = jnp.sum(rs * mi)
        o_ref[s] = jnp.full((8, 128), total, jnp.float32)


def kernel(alignments, input_lengths, output_lengths):
    B, T_out, T_in = alignments.shape
    grid_spec = pltpu.PrefetchScalarGridSpec(
        num_scalar_prefetch=2,
        grid=(B // 2,),
        in_specs=[
            pl.BlockSpec((2, T_out, T_in), lambda b, *_: (b, 0, 0)),
        ],
        out_specs=pl.BlockSpec((2, 8, 128), lambda b, *_: (b, 0, 0)),
        scratch_shapes=[pltpu.VMEM((T_out, 128), jnp.float32)],
    )
    partials = pl.pallas_call(
        functools.partial(_ga_kernel, blk=T_out, t_in=T_in),
        grid_spec=grid_spec,
        out_shape=jax.ShapeDtypeStruct((B, 8, 128), jnp.float32),
        compiler_params=pltpu.CompilerParams(
            dimension_semantics=("arbitrary",),
        ),
    )(input_lengths, output_lengths, alignments)
    return jnp.sum(partials[:, 0, 0]) / B


# speedup vs baseline: 2.5436x; 1.0697x over previous
import functools
import math

import jax
import jax.numpy as jnp
from jax.experimental import pallas as pl
from jax.experimental.pallas import tpu as pltpu

_SIGMA = 0.4
_C = math.sqrt(math.log2(math.e) / (2.0 * _SIGMA * _SIGMA))


def _ga_kernel(in_len_ref, out_len_ref, a_ref, o_ref, is_ref, *, blk, t_in):
    b = pl.program_id(0)

    @pl.when(b == 0)
    def _init_scaled_iota():
        i2 = jax.lax.broadcasted_iota(jnp.int32, (blk, 128), 0)
        is_ref[...] = i2.astype(jnp.float32) * _C

    isc = pltpu.repeat(is_ref[...], t_in // 128, axis=1)
    j_row = jax.lax.broadcasted_iota(jnp.int32, (1, t_in), 1).astype(jnp.float32)
    i_col = jax.lax.broadcasted_iota(jnp.int32, (blk, 1), 0).astype(jnp.float32)

    for s in range(a_ref.shape[0]):
        ti = in_len_ref[4 * b + s].astype(jnp.float32)
        to = out_len_ref[4 * b + s].astype(jnp.float32)
        rc = to / ti * _C

        a = a_ref[s]
        es_row = j_row * rc
        mj = (j_row < ti).astype(jnp.float32)

        u = isc - es_row
        g = jnp.exp2(u * (es_row - isc))
        am = a * mj
        q = am - am * g
        rs = jnp.sum(q, axis=1, keepdims=True)

        mi = (i_col < to).astype(jnp.float32)
        total = jnp.sum(rs * mi)
        o_ref[s] = jnp.full((8, 128), total, jnp.float32)


def kernel(alignments, input_lengths, output_lengths):
    B, T_out, T_in = alignments.shape
    grid_spec = pltpu.PrefetchScalarGridSpec(
        num_scalar_prefetch=2,
        grid=(B // 4,),
        in_specs=[
            pl.BlockSpec((4, T_out, T_in), lambda b, *_: (b, 0, 0)),
        ],
        out_specs=pl.BlockSpec((4, 8, 128), lambda b, *_: (b, 0, 0)),
        scratch_shapes=[pltpu.VMEM((T_out, 128), jnp.float32)],
    )
    partials = pl.pallas_call(
        functools.partial(_ga_kernel, blk=T_out, t_in=T_in),
        grid_spec=grid_spec,
        out_shape=jax.ShapeDtypeStruct((B, 8, 128), jnp.float32),
        compiler_params=pltpu.CompilerParams(
            dimension_semantics=("arbitrary",),
        ),
    )(input_lengths, output_lengths, alignments)
    return jnp.sum(partials[:, 0, 0]) / B


# speedup vs baseline: 2.5679x; 1.0096x over previous
import functools
import math

import jax
import jax.numpy as jnp
from jax.experimental import pallas as pl
from jax.experimental.pallas import tpu as pltpu

_SIGMA = 0.4
_C = math.sqrt(math.log2(math.e) / (2.0 * _SIGMA * _SIGMA))


def _ga_kernel(in_len_ref, out_len_ref, a_ref, o_ref, is_ref, *, blk, t_in):
    b = pl.program_id(0)

    @pl.when(b == 0)
    def _init_scaled_iota():
        i2 = jax.lax.broadcasted_iota(jnp.int32, (blk, 128), 0)
        is_ref[...] = i2.astype(jnp.float32) * _C

    isc = pltpu.repeat(is_ref[...], t_in // 128, axis=1)
    j_row = jax.lax.broadcasted_iota(jnp.int32, (1, t_in), 1).astype(jnp.float32)
    i_col = jax.lax.broadcasted_iota(jnp.int32, (blk, 1), 0).astype(jnp.float32)

    for s in range(a_ref.shape[0]):
        ti = in_len_ref[4 * b + s].astype(jnp.float32)
        to = out_len_ref[4 * b + s].astype(jnp.float32)
        rc = to / ti * _C

        a = a_ref[s]
        es_row = j_row * rc
        mj = (j_row < ti).astype(jnp.float32)

        u = isc - es_row
        g = jnp.exp2(u * (es_row - isc))
        am = a * mj
        q = am - am * g
        rs = jnp.sum(q, axis=1, keepdims=True)

        total = jnp.sum(jnp.where(i_col < to, rs, 0.0))
        o_ref[s] = jnp.full((8, 128), total, jnp.float32)


def kernel(alignments, input_lengths, output_lengths):
    B, T_out, T_in = alignments.shape
    grid_spec = pltpu.PrefetchScalarGridSpec(
        num_scalar_prefetch=2,
        grid=(B // 4,),
        in_specs=[
            pl.BlockSpec((4, T_out, T_in), lambda b, *_: (b, 0, 0)),
        ],
        out_specs=pl.BlockSpec((4, 8, 128), lambda b, *_: (b, 0, 0)),
        scratch_shapes=[pltpu.VMEM((T_out, 128), jnp.float32)],
    )
    partials = pl.pallas_call(
        functools.partial(_ga_kernel, blk=T_out, t_in=T_in),
        grid_spec=grid_spec,
        out_shape=jax.ShapeDtypeStruct((B, 8, 128), jnp.float32),
        compiler_params=pltpu.CompilerParams(
            dimension_semantics=("arbitrary",),
        ),
    )(input_lengths, output_lengths, alignments)
    return jnp.sum(partials[:, 0, 0]) / B


# speedup vs baseline: 2.7152x; 1.0574x over previous
import functools
import math

import jax
import jax.numpy as jnp
from jax.experimental import pallas as pl
from jax.experimental.pallas import tpu as pltpu

_SIGMA = 0.4
_C = math.sqrt(math.log2(math.e) / (2.0 * _SIGMA * _SIGMA))


def _ga_kernel(in_len_ref, out_len_ref, a_ref, o_ref, is_ref, racc_ref, *,
               blk, t_in, nsteps):
    b = pl.program_id(0)

    @pl.when(b == 0)
    def _init():
        i2 = jax.lax.broadcasted_iota(jnp.int32, (blk, 128), 0)
        is_ref[...] = i2.astype(jnp.float32) * _C
        racc_ref[...] = jnp.zeros_like(racc_ref)

    isc = pltpu.repeat(is_ref[...], t_in // 128, axis=1)
    j_row = jax.lax.broadcasted_iota(jnp.int32, (1, t_in), 1).astype(jnp.float32)
    i_col = jax.lax.broadcasted_iota(jnp.int32, (blk, 1), 0).astype(jnp.float32)

    for s in range(a_ref.shape[0]):
        ti = in_len_ref[4 * b + s].astype(jnp.float32)
        to = out_len_ref[4 * b + s].astype(jnp.float32)
        rc = to / ti * _C

        a = a_ref[s]
        es_row = j_row * rc
        mj = (j_row < ti).astype(jnp.float32)

        u = isc - es_row
        g = jnp.exp2(u * (es_row - isc))
        am = a * mj
        q = am - am * g
        rs = jnp.sum(q, axis=1, keepdims=True)
        racc_ref[...] += jnp.where(i_col < to, rs, 0.0)

    @pl.when(b == nsteps - 1)
    def _finalize():
        o_ref[...] = jnp.full((8, 128), jnp.sum(racc_ref[...]), jnp.float32)


def kernel(alignments, input_lengths, output_lengths):
    B, T_out, T_in = alignments.shape
    nsteps = B // 4
    grid_spec = pltpu.PrefetchScalarGridSpec(
        num_scalar_prefetch=2,
        grid=(nsteps,),
        in_specs=[
            pl.BlockSpec((4, T_out, T_in), lambda b, *_: (b, 0, 0)),
        ],
        out_specs=pl.BlockSpec((8, 128), lambda b, *_: (0, 0)),
        scratch_shapes=[
            pltpu.VMEM((T_out, 128), jnp.float32),
            pltpu.VMEM((T_out, 1), jnp.float32),
        ],
    )
    total = pl.pallas_call(
        functools.partial(_ga_kernel, blk=T_out, t_in=T_in, nsteps=nsteps),
        grid_spec=grid_spec,
        out_shape=jax.ShapeDtypeStruct((8, 128), jnp.float32),
        compiler_params=pltpu.CompilerParams(
            dimension_semantics=("arbitrary",),
        ),
    )(input_lengths, output_lengths, alignments)
    return total[0, 0] / B


# speedup vs baseline: 2.7320x; 1.0062x over previous
import functools
import math

import jax
import jax.numpy as jnp
from jax.experimental import pallas as pl
from jax.experimental.pallas import tpu as pltpu

_SIGMA = 0.4
_C = math.sqrt(math.log2(math.e) / (2.0 * _SIGMA * _SIGMA))


def _ga_kernel(in_len_ref, out_len_ref, a_ref, o_ref, is_ref, ic_ref, racc_ref, *,
               blk, t_in, nsteps):
    b = pl.program_id(0)

    @pl.when(b == 0)
    def _init():
        i2 = jax.lax.broadcasted_iota(jnp.int32, (blk, 128), 0)
        is_ref[...] = i2.astype(jnp.float32) * _C
        ic_ref[...] = jax.lax.broadcasted_iota(
            jnp.int32, (blk, 1), 0).astype(jnp.float32)
        racc_ref[...] = jnp.zeros_like(racc_ref)

    isc = pltpu.repeat(is_ref[...], t_in // 128, axis=1)
    j_row = jax.lax.broadcasted_iota(jnp.int32, (1, t_in), 1).astype(jnp.float32)
    i_col = ic_ref[...]

    for s in range(a_ref.shape[0]):
        ti = in_len_ref[4 * b + s].astype(jnp.float32)
        to = out_len_ref[4 * b + s].astype(jnp.float32)
        rc = to / ti * _C

        a = a_ref[s]
        es_row = j_row * rc
        mj = (j_row < ti).astype(jnp.float32)

        u = isc - es_row
        g = jnp.exp2(u * (es_row - isc))
        am = a * mj
        q = am - am * g
        rs = jnp.sum(q, axis=1, keepdims=True)
        racc_ref[...] += jnp.where(i_col < to, rs, 0.0)

    @pl.when(b == nsteps - 1)
    def _finalize():
        o_ref[...] = jnp.full((8, 128), jnp.sum(racc_ref[...]), jnp.float32)


def kernel(alignments, input_lengths, output_lengths):
    B, T_out, T_in = alignments.shape
    nsteps = B // 4
    grid_spec = pltpu.PrefetchScalarGridSpec(
        num_scalar_prefetch=2,
        grid=(nsteps,),
        in_specs=[
            pl.BlockSpec((4, T_out, T_in), lambda b, *_: (b, 0, 0)),
        ],
        out_specs=pl.BlockSpec((8, 128), lambda b, *_: (0, 0)),
        scratch_shapes=[
            pltpu.VMEM((T_out, 128), jnp.float32),
            pltpu.VMEM((T_out, 1), jnp.float32),
            pltpu.VMEM((T_out, 1), jnp.float32),
        ],
    )
    total = pl.pallas_call(
        functools.partial(_ga_kernel, blk=T_out, t_in=T_in, nsteps=nsteps),
        grid_spec=grid_spec,
        out_shape=jax.ShapeDtypeStruct((8, 128), jnp.float32),
        compiler_params=pltpu.CompilerParams(
            dimension_semantics=("arbitrary",),
        ),
    )(input_lengths, output_lengths, alignments)
    return total[0, 0] / B
